# baseline (speedup 1.0000x reference)
import functools

import jax
import jax.numpy as jnp
from jax import lax
from jax.experimental import pallas as pl
from jax.experimental.pallas import tpu as pltpu
from jax.experimental.pallas import tpu_sc as plsc

B, S, D = 8, 2048, 512
N_NODES, K, DK = 64, 32, 64
T = B * N_NODES

NC, NS, L = 2, 16, 16
NW = NC * NS
TPT = T // NW
ST = 256
SC3 = 512


def _energy_body(seq_ref, hid_ref, wpre_ref, bpre_ref, wq_ref, wv_ref,
                 e_ref, e0_ref):
    seq_t = seq_ref[...].reshape(B * ST, D)
    q = jnp.dot(hid_ref[...], wq_ref[...],
                preferred_element_type=jnp.float32)
    pre = jnp.dot(seq_t, wpre_ref[...],
                  preferred_element_type=jnp.float32)
    pre = pre.reshape(B, ST, DK) + bpre_ref[...] + q[:, None, :]
    e_ref[...] = jnp.sum(jnp.tanh(pre) * wv_ref[...], axis=-1)
    e0 = jnp.sum(jnp.tanh(bpre_ref[...] + q) * wv_ref[...], axis=-1)
    e0_ref[...] = jnp.broadcast_to(e0[:, None], (B, 128))


@jax.jit
def _phase1(seq, hidden, wpre, bpre2, wq, wv2):
    return pl.pallas_call(
        _energy_body,
        grid=(S // ST,),
        in_specs=[
            pl.BlockSpec((B, ST, D), lambda s: (0, s, 0)),
            pl.BlockSpec((B, D), lambda s: (0, 0)),
            pl.BlockSpec((D, DK), lambda s: (0, 0)),
            pl.BlockSpec((1, DK), lambda s: (0, 0)),
            pl.BlockSpec((D, DK), lambda s: (0, 0)),
            pl.BlockSpec((1, DK), lambda s: (0, 0)),
        ],
        out_specs=[
            pl.BlockSpec((B, ST), lambda s: (0, s)),
            pl.BlockSpec((B, 128), lambda s: (0, 0)),
        ],
        out_shape=[
            jax.ShapeDtypeStruct((B, S), jnp.float32),
            jax.ShapeDtypeStruct((B, 128), jnp.float32),
        ],
    )(seq, hidden, wpre, bpre2, wq, wv2)


def _sc_body(e_hbm, idx_hbm, len_hbm, a_hbm, e_v, idx_v, len_v, rows_v):
    wid = lax.axis_index("s") * NC + lax.axis_index("c")
    pltpu.sync_copy(e_hbm, e_v)
    pltpu.sync_copy(idx_hbm.at[pl.ds(wid * (TPT * K), TPT * K)], idx_v)
    pltpu.sync_copy(len_hbm.at[pl.ds(wid * TPT, TPT)], len_v)

    zero16 = jnp.zeros((L,), jnp.float32)

    def _zero(j, carry):
        for u in range(8):
            rows_v[pl.ds((j * 8 + u) * L, L)] = zero16
        return carry

    lax.fori_loop(0, TPT * S // (8 * L), _zero, 0)

    kpos = lax.iota(jnp.int32, L)
    lane_eq = [kpos == j for j in range(L)]
    lens = len_v[...]

    for tl in range(TPT):
        t = wid * TPT + tl
        i = t // N_NODES
        ln = lens[tl]
        ens, vlds, kidxs = [], [], []
        for h in range(2):
            kidx = idx_v[pl.ds(tl * K + h * L, L)]
            vld = (kpos + (h * L)) < ln
            gidx = jnp.where(vld, i * S + kidx, B * S + i)
            ens.append(plsc.load_gather(e_v, [gidx]))
            vlds.append(vld)
            kidxs.append(kidx)
        m = jnp.maximum(jnp.max(ens[0]), jnp.max(ens[1]))
        p0 = jnp.exp(ens[0] - m)
        p1 = jnp.exp(ens[1] - m)
        ssum = jnp.full((L,), jnp.sum(p0) + jnp.sum(p1))
        ws = [jnp.where(vlds[0], p0 / ssum, 0.0),
              jnp.where(vlds[1], p1 / ssum, 0.0)]
        for h in range(2):
            tgt = kidxs[h] + (tl * S)
            for j in range(L):
                plsc.addupdate_scatter(rows_v, [tgt], ws[h],
                                       mask=lane_eq[j])

    pltpu.sync_copy(rows_v, a_hbm.at[pl.ds(wid * (TPT * S), TPT * S)])


@jax.jit
def _phase2(e_all, idx_flat, lengths):
    kfn = functools.partial(
        pl.kernel,
        mesh=plsc.VectorSubcoreMesh(core_axis_name="c", subcore_axis_name="s"),
        out_type=jax.ShapeDtypeStruct((T * S,), jnp.float32),
        scratch_types=[
            pltpu.VMEM((B * S + B,), jnp.float32),
            pltpu.VMEM((TPT * K,), jnp.int32),
            pltpu.VMEM((TPT,), jnp.int32),
            pltpu.VMEM((TPT * S,), jnp.float32),
        ],
        compiler_params=pltpu.CompilerParams(needs_layout_passes=False),
    )(_sc_body)
    return kfn(e_all, idx_flat, lengths)


def _combine_body(a_ref, seq_ref, out_ref):
    s = pl.program_id(1)

    @pl.when(s == 0)
    def _():
        out_ref[...] = jnp.zeros_like(out_ref)

    out_ref[0] += jnp.dot(a_ref[0], seq_ref[0],
                          preferred_element_type=jnp.float32)


@jax.jit
def _phase3(a3, seq):
    return pl.pallas_call(
        _combine_body,
        grid=(B, S // SC3),
        in_specs=[
            pl.BlockSpec((1, N_NODES, SC3), lambda i, s: (i, 0, s)),
            pl.BlockSpec((1, SC3, D), lambda i, s: (i, s, 0)),
        ],
        out_specs=pl.BlockSpec((1, N_NODES, D), lambda i, s: (i, 0, 0)),
        out_shape=jax.ShapeDtypeStruct((B, N_NODES, D), jnp.float32),
        compiler_params=pltpu.CompilerParams(
            dimension_semantics=("parallel", "arbitrary")),
    )(a3, seq)


def kernel(seq_output, hidden, index, lengths, W_pre, b_pre, W_q, w_v):
    e, e0p = _phase1(seq_output, hidden, W_pre, b_pre.reshape(1, DK), W_q,
                     w_v.reshape(1, DK))
    e_all = jnp.concatenate([e.reshape(-1), e0p[:, 0]])
    a = _phase2(e_all, index.reshape(-1), lengths)
    nodes = _phase3(a.reshape(B, N_NODES, S), seq_output)
    return (nodes, hidden)


# speedup vs baseline: 1.3579x; 1.3579x over previous
import functools

import jax
import jax.numpy as jnp
from jax import lax
from jax.experimental import pallas as pl
from jax.experimental.pallas import tpu as pltpu
from jax.experimental.pallas import tpu_sc as plsc

B, S, D = 8, 2048, 512
N_NODES, K, DK = 64, 32, 64
T = B * N_NODES

NC, NS, L = 2, 16, 16
NW = NC * NS
TPT = T // NW
SE = S + 128
ST = 256
SC3 = 512


def _energy_body(seq_ref, hid_ref, wpre_ref, bpre_ref, wq_ref, wv_ref,
                 e_ref, e0_ref):
    seq_t = seq_ref[...].reshape(B * ST, D)
    q = jnp.dot(hid_ref[...], wq_ref[...],
                preferred_element_type=jnp.float32)
    pre = jnp.dot(seq_t, wpre_ref[...],
                  preferred_element_type=jnp.float32)
    pre = pre.reshape(B, ST, DK) + bpre_ref[...] + q[:, None, :]
    e_ref[...] = jnp.sum(jnp.tanh(pre) * wv_ref[...], axis=-1)
    e0 = jnp.sum(jnp.tanh(bpre_ref[...] + q) * wv_ref[...], axis=-1)
    e0_ref[...] = jnp.broadcast_to(e0[:, None], (B, 128))


@jax.jit
def _phase1(seq, hidden, wpre, bpre2, wq, wv2):
    return pl.pallas_call(
        _energy_body,
        grid=(S // ST,),
        in_specs=[
            pl.BlockSpec((B, ST, D), lambda s: (0, s, 0)),
            pl.BlockSpec((B, D), lambda s: (0, 0)),
            pl.BlockSpec((D, DK), lambda s: (0, 0)),
            pl.BlockSpec((1, DK), lambda s: (0, 0)),
            pl.BlockSpec((D, DK), lambda s: (0, 0)),
            pl.BlockSpec((1, DK), lambda s: (0, 0)),
        ],
        out_specs=[
            pl.BlockSpec((B, ST), lambda s: (0, s)),
            pl.BlockSpec((B, 128), lambda s: (0, 0)),
        ],
        out_shape=[
            jax.ShapeDtypeStruct((B, S), jnp.float32),
            jax.ShapeDtypeStruct((B, 128), jnp.float32),
        ],
    )(seq, hidden, wpre, bpre2, wq, wv2)


def _sc_body(e_hbm, idx_hbm, len_hbm, a_hbm, e_v, idx_v, len_v, rows_v):
    wid = lax.axis_index("s") * NC + lax.axis_index("c")
    i = wid // (N_NODES // TPT)
    nb = (wid % (N_NODES // TPT)) * TPT
    pltpu.sync_copy(e_hbm.at[i], e_v)
    pltpu.sync_copy(idx_hbm.at[i, pl.ds(nb, TPT)], idx_v)
    pltpu.sync_copy(len_hbm.at[pl.ds(wid * TPT, TPT)], len_v)

    zero16 = jnp.zeros((L,), jnp.float32)

    def _zero(j, carry):
        for r in range(TPT):
            rows_v[r, pl.ds(j * L, L)] = zero16
        return carry

    lax.fori_loop(0, S // L, _zero, 0)

    kpos = lax.iota(jnp.int32, L)
    lens = len_v[...]

    for tl in range(TPT):
        ln = lens[tl]
        row_id = jnp.full((L,), tl, jnp.int32)
        ens, vlds, kidxs = [], [], []
        for h in range(2):
            kidx = idx_v[tl, pl.ds(h * L, L)]
            vld = (kpos + (h * L)) < ln
            gidx = jnp.where(vld, kidx, S)
            ens.append(plsc.load_gather(e_v, [gidx]))
            vlds.append(vld)
            kidxs.append(kidx)
        m = jnp.maximum(jnp.max(ens[0]), jnp.max(ens[1]))
        p0 = jnp.exp(ens[0] - m)
        p1 = jnp.exp(ens[1] - m)
        ssum = jnp.full((L,), jnp.sum(p0) + jnp.sum(p1))
        ws = [jnp.where(vlds[0], p0 / ssum, 0.0),
              jnp.where(vlds[1], p1 / ssum, 0.0)]
        for h in range(2):
            plsc.addupdate_scatter(rows_v, [row_id, kidxs[h]], ws[h])

    pltpu.sync_copy(rows_v, a_hbm.at[i, pl.ds(nb, TPT)])


@jax.jit
def _phase2(e_all, index3, lengths):
    kfn = functools.partial(
        pl.kernel,
        mesh=plsc.VectorSubcoreMesh(core_axis_name="c", subcore_axis_name="s"),
        out_type=jax.ShapeDtypeStruct((B, N_NODES, S), jnp.float32),
        scratch_types=[
            pltpu.VMEM((SE,), jnp.float32),
            pltpu.VMEM((TPT, K), jnp.int32),
            pltpu.VMEM((TPT,), jnp.int32),
            pltpu.VMEM((TPT, S), jnp.float32),
        ],
        compiler_params=pltpu.CompilerParams(needs_layout_passes=False),
    )(_sc_body)
    return kfn(e_all, index3, lengths)


def _combine_body(a_ref, seq_ref, out_ref):
    out_ref[0] = jnp.dot(a_ref[0], seq_ref[0],
                         preferred_element_type=jnp.float32)


@jax.jit
def _phase3(a3, seq):
    return pl.pallas_call(
        _combine_body,
        grid=(B,),
        in_specs=[
            pl.BlockSpec((1, N_NODES, S), lambda i: (i, 0, 0)),
            pl.BlockSpec((1, S, D), lambda i: (i, 0, 0)),
        ],
        out_specs=pl.BlockSpec((1, N_NODES, D), lambda i: (i, 0, 0)),
        out_shape=jax.ShapeDtypeStruct((B, N_NODES, D), jnp.float32),
        compiler_params=pltpu.CompilerParams(
            dimension_semantics=("arbitrary",)),
    )(a3, seq)


def kernel(seq_output, hidden, index, lengths, W_pre, b_pre, W_q, w_v):
    e, e0p = _phase1(seq_output, hidden, W_pre, b_pre.reshape(1, DK), W_q,
                     w_v.reshape(1, DK))
    e_all = jnp.concatenate([e, e0p], axis=1)
    a = _phase2(e_all, index, lengths)
    nodes = _phase3(a, seq_output)
    return (nodes, hidden)
